# Initial kernel scaffold; baseline (speedup 1.0000x reference)
import jax
import jax.numpy as jnp
from jax.experimental import pallas as pl


def kernel(x_sat, x_gateway, x_cell, cell_visibility_matrix, cell_demands, ei_ss, sg_src, sg_dst, gc_src, gc_dst, sc_src, sc_dst, cs_src, cs_dst, params):
    raise NotImplementedError("write your pallas kernel here")


# speedup vs baseline: 4.2717x; 4.2717x over previous
import functools

import jax
import jax.numpy as jnp
from jax import lax
from jax.experimental import pallas as pl
from jax.experimental.pallas import tpu as pltpu
from jax.experimental.pallas import tpu_sc as plsc

N_SAT = 10000
N_GW = 16
N_CELL = 256
D_SAT = 32
HID = 128
ROUNDS = 3
E_SS = 160000
E_SC = 80000
E_GC = 2048

B = 128
SAT_ACC = 10112
CELL_ACC = 384
SAT_PAD_DST = SAT_ACC - 1
CELL_PAD_DST = CELL_ACC - 1
NCH_SS = 79
NCH_SC = 40
ROWS_SAT = SAT_ACC // 16
ROWS_CELL = CELL_ACC // 16

_MESH = plsc.VectorSubcoreMesh(core_axis_name="c", subcore_axis_name="s")

F32 = jnp.float32


def _dot(a, b):
    return lax.dot_general(a, b, (((1,), (0,)), ((), ())),
                           precision=lax.Precision.HIGHEST,
                           preferred_element_type=F32)


@functools.partial(
    pl.kernel,
    out_type=[
        jax.ShapeDtypeStruct((SAT_ACC, HID), F32),
        jax.ShapeDtypeStruct((SAT_ACC, HID), F32),
        jax.ShapeDtypeStruct((CELL_ACC, HID), F32),
    ],
    mesh=_MESH,
    scratch_types=[
        pltpu.VMEM((80, B), jnp.int32),
        pltpu.VMEM((80, B), jnp.int32),
        pltpu.VMEM((B, HID), F32),
        pltpu.VMEM_SHARED((SAT_ACC, HID), F32),
        pltpu.VMEM_SHARED((CELL_ACC, HID), F32),
        pltpu.SemaphoreType.DMA,
    ],
)
def _round_agg(sat_h, cell_h, ssS_h, ssD_h, csS_h, csD_h, scS_h, scD_h, z_h,
               oss, ocs, osc, isrc, idst, rows, acc, acc2, sem):
    c = lax.axis_index("c")
    s = lax.axis_index("s")
    r0 = s * ROWS_SAT
    q0 = s * ROWS_CELL
    pltpu.sync_copy(z_h, acc.at[pl.ds(r0, ROWS_SAT)])

    @pl.when(c == 1)
    def _():
        pltpu.sync_copy(z_h.at[pl.ds(0, ROWS_CELL)], acc2.at[pl.ds(q0, ROWS_CELL)])

    plsc.subcore_barrier()

    @pl.when(c == 0)
    def _():
        pltpu.sync_copy(ssS_h.at[s], isrc.at[pl.ds(0, NCH_SS)])
        pltpu.sync_copy(ssD_h.at[s], idst.at[pl.ds(0, NCH_SS)])

        def body(j, carry):
            pltpu.async_copy(sat_h.at[isrc.at[j]], rows, sem).wait()
            pltpu.sync_copy(rows, acc.at[idst.at[j]], add=True)
            return carry

        lax.fori_loop(0, NCH_SS, body, 0)

    @pl.when(c == 1)
    def _():
        pltpu.sync_copy(csS_h.at[s], isrc.at[pl.ds(0, NCH_SC)])
        pltpu.sync_copy(csD_h.at[s], idst.at[pl.ds(0, NCH_SC)])
        pltpu.sync_copy(scS_h.at[s], isrc.at[pl.ds(NCH_SC, NCH_SC)])
        pltpu.sync_copy(scD_h.at[s], idst.at[pl.ds(NCH_SC, NCH_SC)])

        def body_cs(j, carry):
            pltpu.async_copy(cell_h.at[isrc.at[j]], rows, sem).wait()
            pltpu.sync_copy(rows, acc.at[idst.at[j]], add=True)
            return carry

        lax.fori_loop(0, NCH_SC, body_cs, 0)

        def body_sc(j, carry):
            pltpu.async_copy(sat_h.at[isrc.at[NCH_SC + j]], rows, sem).wait()
            pltpu.sync_copy(rows, acc2.at[idst.at[NCH_SC + j]], add=True)
            return carry

        lax.fori_loop(0, NCH_SC, body_sc, 0)

    plsc.subcore_barrier()

    @pl.when(c == 0)
    def _():
        pltpu.sync_copy(acc.at[pl.ds(r0, ROWS_SAT)], oss.at[pl.ds(r0, ROWS_SAT)])

    @pl.when(c == 1)
    def _():
        pltpu.sync_copy(acc.at[pl.ds(r0, ROWS_SAT)], ocs.at[pl.ds(r0, ROWS_SAT)])
        pltpu.sync_copy(acc2.at[pl.ds(q0, ROWS_CELL)], osc.at[pl.ds(q0, ROWS_CELL)])


@functools.partial(
    pl.kernel,
    out_type=[
        jax.ShapeDtypeStruct((SAT_ACC, HID), F32),
        jax.ShapeDtypeStruct((SAT_ACC, HID), F32),
        jax.ShapeDtypeStruct((CELL_ACC, HID), F32),
        jax.ShapeDtypeStruct((CELL_ACC, HID), F32),
    ],
    mesh=_MESH,
    scratch_types=[
        pltpu.VMEM((81, B), jnp.int32),
        pltpu.VMEM((81, B), jnp.int32),
        pltpu.VMEM((B, HID), F32),
        pltpu.VMEM_SHARED((SAT_ACC, HID), F32),
        pltpu.VMEM_SHARED((CELL_ACC, HID), F32),
        pltpu.VMEM_SHARED((CELL_ACC, HID), F32),
        pltpu.SemaphoreType.DMA,
    ],
)
def _init_agg(xsat_h, xcell_h, xgw_h, ssS_h, ssD_h, csS_h, csD_h, scS_h, scD_h,
              gcS_h, gcD_h, z_h,
              o_ss, o_cs, o_sc, o_gc,
              isrc, idst, rows, acc, acc2, acc3, sem):
    c = lax.axis_index("c")
    s = lax.axis_index("s")
    r0 = s * ROWS_SAT
    q0 = s * ROWS_CELL
    pltpu.sync_copy(z_h, acc.at[pl.ds(r0, ROWS_SAT)])

    @pl.when(c == 1)
    def _():
        pltpu.sync_copy(z_h.at[pl.ds(0, ROWS_CELL)], acc2.at[pl.ds(q0, ROWS_CELL)])
        pltpu.sync_copy(z_h.at[pl.ds(0, ROWS_CELL)], acc3.at[pl.ds(q0, ROWS_CELL)])

    plsc.subcore_barrier()

    @pl.when(c == 0)
    def _():
        pltpu.sync_copy(ssS_h.at[s], isrc.at[pl.ds(0, NCH_SS)])
        pltpu.sync_copy(ssD_h.at[s], idst.at[pl.ds(0, NCH_SS)])

        def body(j, carry):
            pltpu.async_copy(xsat_h.at[isrc.at[j]], rows, sem).wait()
            pltpu.sync_copy(rows, acc.at[idst.at[j]], add=True)
            return carry

        lax.fori_loop(0, NCH_SS, body, 0)

    @pl.when(c == 1)
    def _():
        pltpu.sync_copy(csS_h.at[s], isrc.at[pl.ds(0, NCH_SC)])
        pltpu.sync_copy(csD_h.at[s], idst.at[pl.ds(0, NCH_SC)])
        pltpu.sync_copy(scS_h.at[s], isrc.at[pl.ds(NCH_SC, NCH_SC)])
        pltpu.sync_copy(scD_h.at[s], idst.at[pl.ds(NCH_SC, NCH_SC)])
        pltpu.sync_copy(gcS_h.at[s], isrc.at[pl.ds(2 * NCH_SC, 1)])
        pltpu.sync_copy(gcD_h.at[s], idst.at[pl.ds(2 * NCH_SC, 1)])

        def body_cs(j, carry):
            pltpu.async_copy(xcell_h.at[isrc.at[j]], rows, sem).wait()
            pltpu.sync_copy(rows, acc.at[idst.at[j]], add=True)
            return carry

        lax.fori_loop(0, NCH_SC, body_cs, 0)

        def body_sc(j, carry):
            pltpu.async_copy(xsat_h.at[isrc.at[NCH_SC + j]], rows, sem).wait()
            pltpu.sync_copy(rows, acc2.at[idst.at[NCH_SC + j]], add=True)
            return carry

        lax.fori_loop(0, NCH_SC, body_sc, 0)

        pltpu.async_copy(xgw_h.at[isrc.at[2 * NCH_SC]], rows, sem).wait()
        pltpu.sync_copy(rows, acc3.at[idst.at[2 * NCH_SC]], add=True)

    plsc.subcore_barrier()

    @pl.when(c == 0)
    def _():
        pltpu.sync_copy(acc.at[pl.ds(r0, ROWS_SAT)], o_ss.at[pl.ds(r0, ROWS_SAT)])

    @pl.when(c == 1)
    def _():
        pltpu.sync_copy(acc.at[pl.ds(r0, ROWS_SAT)], o_cs.at[pl.ds(r0, ROWS_SAT)])
        pltpu.sync_copy(acc2.at[pl.ds(q0, ROWS_CELL)], o_sc.at[pl.ds(q0, ROWS_CELL)])
        pltpu.sync_copy(acc3.at[pl.ds(q0, ROWS_CELL)], o_gc.at[pl.ds(q0, ROWS_CELL)])


_BLK = 1000
_GRID = N_SAT // _BLK


def _full(shape):
    return pl.BlockSpec(shape, lambda i: (0, 0))


def _rows(shape):
    return pl.BlockSpec(shape, lambda i: (i, 0))


def _fold_body(wlss_r, wrss_r, wrcs_r, wlsc_r, wg_r, wc_r, bg_r, bc_r,
               blss_r, blcs_r, blsc_r,
               u1_r, u2_r, v1_r, u3_r, u4_r, v2_r):
    wg = wg_r[...]
    wc = wc_r[...]
    bg = bg_r[...]
    bc = bc_r[...]

    def fold_m(m):
        return m[0:HID] + _dot(wg, m[HID:HID + N_GW]) + _dot(wc, m[HID + N_GW:])

    def fold_v(m):
        return _dot(bg, m[HID:HID + N_GW]) + _dot(bc, m[HID + N_GW:])

    wlss = wlss_r[...]
    wr = wrss_r[...] + wrcs_r[...]
    wlsc = wlsc_r[...]
    u1_r[...] = fold_m(wlss)
    u2_r[...] = fold_m(wr)
    v1_r[...] = fold_m(wlsc)
    u3_r[...] = fold_v(wlss)
    u4_r[...] = blss_r[...] + blcs_r[...] + fold_v(wr)
    v2_r[...] = fold_v(wlsc) + blsc_r[...]


_fold = pl.pallas_call(
    _fold_body,
    out_shape=[jax.ShapeDtypeStruct((HID, HID), F32),
               jax.ShapeDtypeStruct((HID, HID), F32),
               jax.ShapeDtypeStruct((HID, HID), F32),
               jax.ShapeDtypeStruct((1, HID), F32),
               jax.ShapeDtypeStruct((1, HID), F32),
               jax.ShapeDtypeStruct((1, HID), F32)],
)


def _round_dense_body(sat_r, sss_r, css_r, scs_r, ccs_r, vis_r,
                      u1_r, wlcs_r, u2_r, u3_r, u4_r,
                      wg_r, bg_r, wc_r, bc_r, demw_r,
                      ns_r, gl_r, cl_r, dem_r):
    css = css_r[...]
    ccs = ccs_r[...]
    m_ss = sss_r[...] * (1.0 / jnp.maximum(css, 1.0))
    m_cs = scs_r[...] * (1.0 / jnp.maximum(ccs, 1.0))
    ind = (css > 0.0).astype(F32)
    z = (_dot(m_ss, u1_r[...]) + _dot(m_cs, wlcs_r[...])
         + _dot(sat_r[...], u2_r[...]) + ind * u3_r[...] + u4_r[...])
    ns = jnp.maximum(z, 0.0)
    ns_r[...] = ns
    gl_r[...] = _dot(ns, wg_r[...]) + bg_r[...]
    cl = _dot(ns, wc_r[...]) + bc_r[...]
    cl_r[...] = cl
    sig = 1.0 / (1.0 + jnp.exp(-cl))
    dem_r[...] = jnp.sum(sig * vis_r[...] * demw_r[...], axis=1, keepdims=True)


_round_dense = pl.pallas_call(
    _round_dense_body,
    grid=(_GRID,),
    in_specs=[_rows((_BLK, HID)), _rows((_BLK, HID)), _rows((_BLK, 1)),
              _rows((_BLK, HID)), _rows((_BLK, 1)), _rows((_BLK, N_CELL)),
              _full((HID, HID)), _full((HID, HID)), _full((HID, HID)),
              _full((1, HID)), _full((1, HID)),
              _full((HID, N_GW)), _full((1, N_GW)),
              _full((HID, N_CELL)), _full((1, N_CELL)), _full((1, N_CELL))],
    out_specs=[_rows((_BLK, HID)), _rows((_BLK, N_GW)),
               _rows((_BLK, N_CELL)), _rows((_BLK, 1))],
    out_shape=[jax.ShapeDtypeStruct((N_SAT, HID), F32),
               jax.ShapeDtypeStruct((N_SAT, N_GW), F32),
               jax.ShapeDtypeStruct((N_SAT, N_CELL), F32),
               jax.ShapeDtypeStruct((N_SAT, 1), F32)],
)


def _cell_round_body(ssc_r, csc_r, cell_r, v1_r, v2_r, wrsc_r, out_r):
    csc = csc_r[...]
    m = ssc_r[...] * (1.0 / jnp.maximum(csc, 1.0))
    ind = (csc > 0.0).astype(F32)
    out_r[...] = jnp.maximum(
        _dot(m, v1_r[...]) + ind * v2_r[...] + _dot(cell_r[...], wrsc_r[...]), 0.0)


_cell_round = pl.pallas_call(
    _cell_round_body,
    out_shape=jax.ShapeDtypeStruct((N_CELL, HID), F32),
)


def _init_dense_body(xs_r, sss_r, css_r, scs_r, ccs_r, vis_r,
                     wla_r, wlb_r, wra_r, wrb_r, bla_r, blb_r,
                     wg_r, bg_r, wc_r, bc_r, demw_r,
                     sat_r, gl_r, cl_r, dem_r):
    css = css_r[...]
    ccs = ccs_r[...]
    m_ss = sss_r[...] * (1.0 / jnp.maximum(css, 1.0))
    m_cs = scs_r[...] * (1.0 / jnp.maximum(ccs, 1.0))
    z = (_dot(m_ss, wla_r[...]) + _dot(m_cs, wlb_r[...])
         + _dot(xs_r[...], wra_r[...] + wrb_r[...]) + bla_r[...] + blb_r[...])
    sat = jnp.maximum(z, 0.0)
    sat_r[...] = sat
    gl_r[...] = _dot(sat, wg_r[...]) + bg_r[...]
    cl = _dot(sat, wc_r[...]) + bc_r[...]
    cl_r[...] = cl
    sig = 1.0 / (1.0 + jnp.exp(-cl))
    dem_r[...] = jnp.sum(sig * vis_r[...] * demw_r[...], axis=1, keepdims=True)


_init_dense = pl.pallas_call(
    _init_dense_body,
    grid=(_GRID,),
    in_specs=[_rows((_BLK, D_SAT)), _rows((_BLK, D_SAT)), _rows((_BLK, 1)),
              _rows((_BLK, 16)), _rows((_BLK, 1)), _rows((_BLK, N_CELL)),
              _full((D_SAT, HID)), _full((16, HID)),
              _full((D_SAT, HID)), _full((D_SAT, HID)),
              _full((1, HID)), _full((1, HID)),
              _full((HID, N_GW)), _full((1, N_GW)),
              _full((HID, N_CELL)), _full((1, N_CELL)), _full((1, N_CELL))],
    out_specs=[_rows((_BLK, HID)), _rows((_BLK, N_GW)),
               _rows((_BLK, N_CELL)), _rows((_BLK, 1))],
    out_shape=[jax.ShapeDtypeStruct((N_SAT, HID), F32),
               jax.ShapeDtypeStruct((N_SAT, N_GW), F32),
               jax.ShapeDtypeStruct((N_SAT, N_CELL), F32),
               jax.ShapeDtypeStruct((N_SAT, 1), F32)],
)


def _init_cell_body(sgc_r, cgc_r, ssc_r, csc_r, xc_r,
                    wlc_r, wld_r, wrc_r, wrd_r, blc_r, bld_r, out_r):
    cgc = cgc_r[...]
    csc = csc_r[...]
    m_gc = sgc_r[...] * (1.0 / jnp.maximum(cgc, 1.0))
    m_sc = ssc_r[...] * (1.0 / jnp.maximum(csc, 1.0))
    z = (_dot(m_gc, wlc_r[...]) + _dot(m_sc, wld_r[...])
         + _dot(xc_r[...], wrc_r[...] + wrd_r[...]) + blc_r[...] + bld_r[...])
    out_r[...] = jnp.maximum(z, 0.0)


_init_cell = pl.pallas_call(
    _init_cell_body,
    out_shape=jax.ShapeDtypeStruct((N_CELL, HID), F32),
)


def _prep_idx(idx, nch, fill):
    cap = 16 * nch * B
    idx = idx.astype(jnp.int32)
    pad = cap - idx.shape[0]
    if pad:
        idx = jnp.concatenate([idx, jnp.full((pad,), fill, jnp.int32)])
    return idx.reshape(16, nch, B)


def kernel(x_sat, x_gateway, x_cell, cell_visibility_matrix, cell_demands,
           ei_ss, sg_src, sg_dst, gc_src, gc_dst, sc_src, sc_dst,
           cs_src, cs_dst, params):
    p = params
    vis = cell_visibility_matrix
    demw = cell_demands[None, :]

    def _pad_table(x):
        n, d = x.shape
        return jnp.concatenate(
            [x, jnp.zeros((n, HID - 1 - d), F32), jnp.ones((n, 1), F32)], axis=1)

    xs128 = _pad_table(x_sat)
    xc128 = _pad_table(x_cell)
    xg128 = _pad_table(x_gateway)

    ssS = _prep_idx(ei_ss[0], NCH_SS, 0)
    ssD = _prep_idx(ei_ss[1], NCH_SS, SAT_PAD_DST)
    csS = _prep_idx(cs_src, NCH_SC, 0)
    csD = _prep_idx(cs_dst, NCH_SC, SAT_PAD_DST)
    scS = _prep_idx(sc_src, NCH_SC, 0)
    scD = _prep_idx(sc_dst, NCH_SC, CELL_PAD_DST)
    gcS = _prep_idx(gc_src, 1, 0)
    gcD = _prep_idx(gc_dst, 1, CELL_PAD_DST)

    z128 = jnp.zeros((ROWS_SAT, HID), F32)

    S_ss0, S_cs0, S_sc0, S_gc0 = _init_agg(
        xs128, xc128, xg128, ssS, ssD, csS, csD, scS, scD, gcS, gcD, z128)

    cnt_ss = S_ss0[:N_SAT, HID - 1:]
    cnt_cs = S_cs0[:N_SAT, HID - 1:]
    cnt_sc = S_sc0[:N_CELL, HID - 1:]
    cnt_gc = S_gc0[:N_CELL, HID - 1:]

    sat, gl, cl, dem = _init_dense(
        x_sat, S_ss0[:N_SAT, :D_SAT], cnt_ss, S_cs0[:N_SAT, :16], cnt_cs, vis,
        p["init_ss_Wl"], jnp.pad(p["init_cs_Wl"], ((0, 8), (0, 0))),
        p["init_ss_Wr"], p["init_cs_Wr"],
        p["init_ss_bl"][None, :], p["init_cs_bl"][None, :],
        p["gw_head_W0"], p["gw_head_b0"][None, :],
        p["cell_head_W0"], p["cell_head_b0"][None, :], demw)

    cell = _init_cell(
        S_gc0[:N_CELL, :16], cnt_gc, S_sc0[:N_CELL, :D_SAT], cnt_sc, x_cell,
        jnp.pad(p["init_gc_Wl"], ((0, 8), (0, 0))), p["init_sc_Wl"],
        p["init_gc_Wr"], p["init_sc_Wr"],
        p["init_gc_bl"][None, :], p["init_sc_bl"][None, :])

    gls, cls, dems = [gl], [cl], [dem]
    for i in range(ROUNDS):
        u1, u2, v1, u3, u4, v2 = _fold(
            p["r%d_ss_Wl" % i], p["r%d_ss_Wr" % i], p["r%d_cs_Wr" % i],
            p["r%d_sc_Wl" % i],
            p["gw_head_W%d" % i], p["cell_head_W%d" % i],
            p["gw_head_b%d" % i][None, :], p["cell_head_b%d" % i][None, :],
            p["r%d_ss_bl" % i][None, :], p["r%d_cs_bl" % i][None, :],
            p["r%d_sc_bl" % i][None, :])

        S_ss, S_cs, S_sc = _round_agg(sat, cell, ssS, ssD, csS, csD, scS, scD,
                                      z128)

        new_sat, gl, cl, dem = _round_dense(
            sat, S_ss[:N_SAT], cnt_ss, S_cs[:N_SAT], cnt_cs, vis,
            u1, p["r%d_cs_Wl" % i], u2, u3, u4,
            p["gw_head_W%d" % (i + 1)], p["gw_head_b%d" % (i + 1)][None, :],
            p["cell_head_W%d" % (i + 1)], p["cell_head_b%d" % (i + 1)][None, :],
            demw)
        cell = _cell_round(S_sc[:N_CELL], cnt_sc, cell, v1, v2,
                           p["r%d_sc_Wr" % i])
        sat = new_sat
        gls.append(gl)
        cls.append(cl)
        dems.append(dem)

    return (jnp.stack(gls), jnp.stack(cls),
            jnp.stack([d[:, 0] for d in dems]))
